# Initial kernel scaffold; baseline (speedup 1.0000x reference)
import functools

import jax
import jax.numpy as jnp
from jax import lax
from jax.experimental import pallas as pl
from jax.experimental.pallas import tpu as pltpu
from jax.experimental.pallas import tpu_sc as plsc

N = 10000
E = 320000
F = 128
HF = 64
EPS = 1e-7
BN_EPS = 1e-5

NC = 2
NS = 16
LANES = 16

EPT = E // NS
B = 80
NCHUNK = EPT // B
NPT = N // NS
FB = 125
NFIN = NPT // FB


def _agg_body(h2, src_hbm, dst_hbm, out,
              acc_ex, acc_mex,
              src_v, dst_v, gidx_v, rows_v, ex_v, mex_v,
              exb, mexb, outb, sem):
    c = lax.axis_index("c")
    s = lax.axis_index("s")

    @pl.loop(0, FB)
    def _zero(i):
        for k in range(HF // LANES):
            outb[i, pl.ds(k * LANES, LANES)] = jnp.zeros((LANES,), jnp.float32)

    @pl.loop(0, NFIN)
    def _zcopy(p):
        nb = s * NPT + p * FB
        pltpu.sync_copy(outb, acc_ex.at[pl.ds(nb, FB)])
        pltpu.sync_copy(outb, acc_mex.at[pl.ds(nb, FB)])

    plsc.subcore_barrier()

    @pl.loop(0, NCHUNK)
    def _chunk(g):
        e0 = s * EPT + g * B
        pltpu.sync_copy(src_hbm.at[pl.ds(e0, B)], src_v)
        pltpu.sync_copy(dst_hbm.at[pl.ds(e0, B)], dst_v)

        @pl.loop(0, B // LANES)
        def _gidx(k):
            sv = src_v[pl.ds(k * LANES, LANES)]
            gidx_v[pl.ds(k * LANES, LANES)] = sv * 2 + c

        pltpu.async_copy(h2.at[gidx_v], rows_v, sem).wait()

        @pl.loop(0, B)
        def _edge(i):
            for k in range(HF // LANES):
                v = rows_v[i, pl.ds(k * LANES, LANES)]
                m = jnp.maximum(v, 0.0) + EPS
                e = jnp.exp(m)
                ex_v[i, pl.ds(k * LANES, LANES)] = e
                mex_v[i, pl.ds(k * LANES, LANES)] = m * e

        pltpu.sync_copy(ex_v, acc_ex.at[dst_v], add=True)
        pltpu.sync_copy(mex_v, acc_mex.at[dst_v], add=True)

    plsc.subcore_barrier()

    @pl.loop(0, NFIN)
    def _fin(p):
        nb = s * NPT + p * FB
        pltpu.sync_copy(acc_ex.at[pl.ds(nb, FB)], exb)
        pltpu.sync_copy(acc_mex.at[pl.ds(nb, FB)], mexb)

        @pl.loop(0, FB)
        def _node(i):
            for k in range(HF // LANES):
                d = exb[i, pl.ds(k * LANES, LANES)]
                u = mexb[i, pl.ds(k * LANES, LANES)]
                outb[i, pl.ds(k * LANES, LANES)] = u / (d + 1e-16)

        pltpu.sync_copy(outb, out.at[c, pl.ds(nb, FB)])


_sc_aggregate = pl.kernel(
    _agg_body,
    out_type=jax.ShapeDtypeStruct((NC, N, HF), jnp.float32),
    mesh=plsc.VectorSubcoreMesh(core_axis_name="c", subcore_axis_name="s"),
    scratch_types=[
        pltpu.VMEM_SHARED((N, HF), jnp.float32),
        pltpu.VMEM_SHARED((N, HF), jnp.float32),
        pltpu.VMEM((B,), jnp.int32),
        pltpu.VMEM((B,), jnp.int32),
        pltpu.VMEM((B,), jnp.int32),
        pltpu.VMEM((B, HF), jnp.float32),
        pltpu.VMEM((B, HF), jnp.float32),
        pltpu.VMEM((B, HF), jnp.float32),
        pltpu.VMEM((FB, HF), jnp.float32),
        pltpu.VMEM((FB, HF), jnp.float32),
        pltpu.VMEM((FB, HF), jnp.float32),
        pltpu.SemaphoreType.DMA,
    ],
)


def _mlp_body(final, aggr_ref, h_ref, wa_ref, ba_ref, gs_ref, be_ref,
              wb_ref, bb_ref, o_ref):
    a = jnp.concatenate([aggr_ref[0], aggr_ref[1]], axis=-1) + h_ref[...]
    t = jnp.dot(a, wa_ref[...], preferred_element_type=jnp.float32)
    t = t + ba_ref[...]
    t = gs_ref[...] * (t * (1.0 / jnp.sqrt(1.0 + BN_EPS))) + be_ref[...]
    t = jnp.maximum(t, 0.0)
    o = jnp.dot(t, wb_ref[...], preferred_element_type=jnp.float32)
    o = o + bb_ref[...]
    if final == "relu":
        o_ref[...] = jnp.maximum(o, 0.0)
    else:
        m = jnp.max(o, axis=1, keepdims=True)
        ex = jnp.exp(o - m)
        lse = jnp.log(jnp.sum(ex, axis=1, keepdims=True)) + m
        o_ref[...] = o - lse


def _mlp(aggr, h, wa, ba, g, be, wb, bb, final):
    bn = 1000
    fmid = wa.shape[1]
    fout = wb.shape[1]
    grid = (N // bn,)
    return pl.pallas_call(
        functools.partial(_mlp_body, final),
        grid=grid,
        in_specs=[
            pl.BlockSpec((NC, bn, HF), lambda i: (0, i, 0)),
            pl.BlockSpec((bn, F), lambda i: (i, 0)),
            pl.BlockSpec((F, fmid), lambda i: (0, 0)),
            pl.BlockSpec((1, fmid), lambda i: (0, 0)),
            pl.BlockSpec((1, fmid), lambda i: (0, 0)),
            pl.BlockSpec((1, fmid), lambda i: (0, 0)),
            pl.BlockSpec((fmid, fout), lambda i: (0, 0)),
            pl.BlockSpec((1, fout), lambda i: (0, 0)),
        ],
        out_specs=pl.BlockSpec((bn, fout), lambda i: (i, 0)),
        out_shape=jax.ShapeDtypeStruct((N, fout), jnp.float32),
    )(aggr, h, wa, ba.reshape(1, -1), g.reshape(1, -1), be.reshape(1, -1),
      wb, bb.reshape(1, -1))


def kernel(x, edge_index, W1a, b1a, g1, be1, W1b, b1b,
           W2a, b2a, g2, be2, W2b, b2b):
    src = edge_index[0]
    dst = edge_index[1]
    aggr1 = _sc_aggregate(x.reshape(2 * N, HF), src, dst)
    h1 = _mlp(aggr1, x, W1a, b1a, g1, be1, W1b, b1b, final="relu")
    aggr2 = _sc_aggregate(h1.reshape(2 * N, HF), src, dst)
    return _mlp(aggr2, h1, W2a, b2a, g2, be2, W2b, b2b, final="logsoftmax")


# speedup vs baseline: 6.7403x; 6.7403x over previous
import functools

import jax
import jax.numpy as jnp
from jax import lax
from jax.experimental import pallas as pl
from jax.experimental.pallas import tpu as pltpu
from jax.experimental.pallas import tpu_sc as plsc

N = 10000
E = 320000
F = 128
HF = 64
EPS = 1e-7
BN_EPS = 1e-5

NC = 2
NS = 16
LANES = 16

EPT = E // NS
B = 80
NCHUNK = EPT // B
NP = 10240
NPT = NP // NS
FB = 128
NFIN = NPT // FB


def _agg_body(h2, src_hbm, dst_hbm, out,
              acc_ex, acc_mex,
              src_v, dst_v, gidx_v, rows_v, ex_v, mex_v,
              exb, mexb, outb, sem):
    c = lax.axis_index("c")
    s = lax.axis_index("s")

    @pl.loop(0, FB)
    def _zero(i):
        for k in range(HF // LANES):
            outb[i, pl.ds(k * LANES, LANES)] = jnp.zeros((LANES,), jnp.float32)

    @pl.loop(0, NFIN)
    def _zcopy(p):
        nb = s * NPT + p * FB
        pltpu.sync_copy(outb, acc_ex.at[pl.ds(nb, FB)])
        pltpu.sync_copy(outb, acc_mex.at[pl.ds(nb, FB)])

    plsc.subcore_barrier()

    @pl.loop(0, NCHUNK)
    def _chunk(g):
        e0 = s * EPT + g * B
        pltpu.sync_copy(src_hbm.at[pl.ds(e0, B)], src_v)
        pltpu.sync_copy(dst_hbm.at[pl.ds(e0, B)], dst_v)

        @pl.loop(0, B // LANES)
        def _gidx(k):
            sv = src_v[pl.ds(k * LANES, LANES)]
            gidx_v[pl.ds(k * LANES, LANES)] = sv * 2 + c

        pltpu.async_copy(h2.at[gidx_v], rows_v, sem).wait()

        @pl.loop(0, B)
        def _edge(i):
            for k in range(HF // LANES):
                v = rows_v[i, pl.ds(k * LANES, LANES)]
                m = jnp.maximum(v, 0.0) + EPS
                e = jnp.exp(m)
                ex_v[i, pl.ds(k * LANES, LANES)] = e
                mex_v[i, pl.ds(k * LANES, LANES)] = m * e

        pltpu.sync_copy(ex_v, acc_ex.at[dst_v], add=True)
        pltpu.sync_copy(mex_v, acc_mex.at[dst_v], add=True)

    plsc.subcore_barrier()

    @pl.loop(0, NFIN)
    def _fin(p):
        nb = s * NPT + p * FB
        pltpu.sync_copy(acc_ex.at[pl.ds(nb, FB)], exb)
        pltpu.sync_copy(acc_mex.at[pl.ds(nb, FB)], mexb)

        @pl.loop(0, FB)
        def _node(i):
            for k in range(HF // LANES):
                d = exb[i, pl.ds(k * LANES, LANES)]
                u = mexb[i, pl.ds(k * LANES, LANES)]
                outb[i, pl.ds(k * LANES, LANES)] = u / (d + 1e-16)

        pltpu.sync_copy(outb, out.at[c, pl.ds(nb, FB)])


_sc_aggregate = pl.kernel(
    _agg_body,
    out_type=jax.ShapeDtypeStruct((NC, NP, HF), jnp.float32),
    mesh=plsc.VectorSubcoreMesh(core_axis_name="c", subcore_axis_name="s"),
    compiler_params=pltpu.CompilerParams(use_tc_tiling_on_sc=False),
    scratch_types=[
        pltpu.VMEM_SHARED((NP, HF), jnp.float32),
        pltpu.VMEM_SHARED((NP, HF), jnp.float32),
        pltpu.VMEM((B,), jnp.int32),
        pltpu.VMEM((B,), jnp.int32),
        pltpu.VMEM((B,), jnp.int32),
        pltpu.VMEM((B, HF), jnp.float32),
        pltpu.VMEM((B, HF), jnp.float32),
        pltpu.VMEM((B, HF), jnp.float32),
        pltpu.VMEM((FB, HF), jnp.float32),
        pltpu.VMEM((FB, HF), jnp.float32),
        pltpu.VMEM((FB, HF), jnp.float32),
        pltpu.SemaphoreType.DMA,
    ],
)


def _mlp_body(final, aggr_ref, h_ref, wa_ref, ba_ref, gs_ref, be_ref,
              wb_ref, bb_ref, o_ref):
    a = jnp.concatenate([aggr_ref[0], aggr_ref[1]], axis=-1) + h_ref[...]
    t = jnp.dot(a, wa_ref[...], preferred_element_type=jnp.float32)
    t = t + ba_ref[...]
    t = gs_ref[...] * (t * (1.0 / jnp.sqrt(1.0 + BN_EPS))) + be_ref[...]
    t = jnp.maximum(t, 0.0)
    o = jnp.dot(t, wb_ref[...], preferred_element_type=jnp.float32)
    o = o + bb_ref[...]
    if final == "relu":
        o_ref[...] = jnp.maximum(o, 0.0)
    else:
        m = jnp.max(o, axis=1, keepdims=True)
        ex = jnp.exp(o - m)
        lse = jnp.log(jnp.sum(ex, axis=1, keepdims=True)) + m
        o_ref[...] = o - lse


def _mlp(aggr, h, wa, ba, g, be, wb, bb, final):
    bn = 1000
    fmid = wa.shape[1]
    fout = wb.shape[1]
    grid = (N // bn,)
    return pl.pallas_call(
        functools.partial(_mlp_body, final),
        grid=grid,
        in_specs=[
            pl.BlockSpec((NC, bn, HF), lambda i: (0, i, 0)),
            pl.BlockSpec((bn, F), lambda i: (i, 0)),
            pl.BlockSpec((F, fmid), lambda i: (0, 0)),
            pl.BlockSpec((1, fmid), lambda i: (0, 0)),
            pl.BlockSpec((1, fmid), lambda i: (0, 0)),
            pl.BlockSpec((1, fmid), lambda i: (0, 0)),
            pl.BlockSpec((fmid, fout), lambda i: (0, 0)),
            pl.BlockSpec((1, fout), lambda i: (0, 0)),
        ],
        out_specs=pl.BlockSpec((bn, fout), lambda i: (i, 0)),
        out_shape=jax.ShapeDtypeStruct((N, fout), jnp.float32),
    )(aggr, h, wa, ba.reshape(1, -1), g.reshape(1, -1), be.reshape(1, -1),
      wb, bb.reshape(1, -1))


def kernel(x, edge_index, W1a, b1a, g1, be1, W1b, b1b,
           W2a, b2a, g2, be2, W2b, b2b):
    src = edge_index[0]
    dst = edge_index[1]
    aggr1 = _sc_aggregate(x.reshape(2 * N, HF), src, dst)[:, :N]
    h1 = _mlp(aggr1, x, W1a, b1a, g1, be1, W1b, b1b, final="relu")
    aggr2 = _sc_aggregate(h1.reshape(2 * N, HF), src, dst)[:, :N]
    return _mlp(aggr2, h1, W2a, b2a, g2, be2, W2b, b2b, final="logsoftmax")
